# baseline (speedup 1.0000x reference)
import jax
import jax.numpy as jnp
from jax.experimental import pallas as pl
from jax.experimental.pallas import tpu as pltpu


def _make_body(bs, t):
    nsplit = 4 if t % 4 == 0 else 1
    h = t // nsplit
    spans = [(i * h, h) for i in range(nsplit)]

    def body(emb_ref, out_ref, vmem, lsem, wsem):
        loads = [
            pltpu.make_async_copy(
                emb_ref.at[pl.ds(o, n)], vmem.at[pl.ds(o, n)], lsem
            )
            for (o, n) in spans
        ]
        for l in loads:
            l.start()
        writes = []
        for i, (o, n) in enumerate(spans):
            loads[i].wait()
            half = [
                pltpu.make_async_copy(
                    vmem.at[pl.ds(o, n)],
                    out_ref.at[b, pl.ds(o, n)],
                    wsem,
                )
                for b in range(bs)
            ]
            for c in half:
                c.start()
            writes.extend(half)
        for c in writes:
            c.wait()

    return body


def kernel(mask, embed_weight):
    bs, t = mask.shape
    n_embed, d = embed_weight.shape
    emb = embed_weight[:t]

    out = pl.pallas_call(
        _make_body(bs, t),
        in_specs=[pl.BlockSpec(memory_space=pl.ANY)],
        out_specs=pl.BlockSpec(memory_space=pl.ANY),
        out_shape=jax.ShapeDtypeStruct((bs, t, d), embed_weight.dtype),
        scratch_shapes=[
            pltpu.VMEM((t, d), embed_weight.dtype),
            pltpu.SemaphoreType.DMA,
            pltpu.SemaphoreType.DMA,
        ],
    )(emb)
    return out
